# baseline (speedup 1.0000x reference)
import jax
import jax.numpy as jnp
from jax.experimental import pallas as pl
from jax.experimental.pallas import tpu as pltpu

_S = 2048
_D = 2048
_TN = 128
_TM = 256


def _fused_kernel(ctx_ref, mem_ref, ew_ref,
                  wc_ref, bc_ref, wb_ref, bb_ref,
                  wr_ref, br_ref, wd_ref, bd_ref,
                  out_ref, mean_ref, coef_ref):
    n = pl.program_id(0)

    @pl.when(n == 0)
    def _routing():
        v = ew_ref[:]
        cols = [v[:, e:e + 1] for e in range(4)]
        ws = []
        for e in range(4):
            rank = jnp.zeros_like(cols[e], dtype=jnp.int32)
            for f in range(4):
                if f == e:
                    continue
                if f < e:
                    beats = cols[f] >= cols[e]
                else:
                    beats = cols[f] > cols[e]
                rank = rank + beats.astype(jnp.int32)
            sel = (rank < 2).astype(jnp.float32)
            ws.append(cols[e] * sel)
        s = ws[0] + ws[1] + ws[2] + ws[3]
        inv = 1.0 / jnp.maximum(s, 1e-8)
        w = [wi * inv for wi in ws]
        alphas = (0.4, 1.0, 0.85, 1.15)
        s_w = w[0] + w[1] + w[2] + w[3]
        coef_ref[:, 0:1] = alphas[0] * w[0]
        coef_ref[:, 1:2] = alphas[1] * w[1]
        coef_ref[:, 2:3] = alphas[2] * w[2]
        coef_ref[:, 3:4] = alphas[3] * w[3]
        coef_ref[:, 4:5] = s_w
        coef_ref[:, 5:6] = 0.15 * w[2]
        coef_ref[:, 6:8] = jnp.zeros((_S, 2), jnp.float32)
        wcat = jnp.concatenate(w, axis=1)
        m = jnp.sum(wcat, axis=0, keepdims=True) / _S
        mrow = jnp.concatenate([m, jnp.zeros((1, 124), jnp.float32)], axis=1)
        mean_ref[:] = jnp.concatenate(
            [mrow, jnp.zeros((7, 128), jnp.float32)], axis=0)

    wtops = []
    wbots = []
    bias = []
    for w_ref, b_ref in ((wc_ref, bc_ref), (wb_ref, bb_ref),
                         (wr_ref, br_ref), (wd_ref, bd_ref)):
        wtops.append(w_ref[0:_D, :].astype(jnp.bfloat16))
        wbots.append(w_ref[_D:2 * _D, :].astype(jnp.bfloat16))
        bias.append(b_ref[:])

    col = pl.ds(n * _TN, _TN)
    for i in range(_S // _TM):
        rows = pl.ds(i * _TM, _TM)
        cb = ctx_ref[rows, :]
        mb = mem_ref[rows, :]
        acc = jnp.zeros((_TM, _TN), jnp.float32)
        for e in range(4):
            h = jnp.dot(cb, wtops[e], preferred_element_type=jnp.float32)
            h = h + jnp.dot(mb, wbots[e], preferred_element_type=jnp.float32)
            gate = jax.nn.sigmoid(h + bias[e])
            acc = acc + coef_ref[rows, e:e + 1] * gate
        ctx32 = ctx_ref[rows, col].astype(jnp.float32)
        mem32 = mem_ref[rows, col].astype(jnp.float32)
        out_ref[rows, :] = (coef_ref[rows, 4:5] * ctx32
                            + (mem32 - ctx32) * acc
                            + coef_ref[rows, 5:6] * mem32)


@jax.jit
def kernel(context_state, memory_state, expert_weights,
           W_conservative, b_conservative, W_base, b_base,
           W_bridge, b_bridge, W_memory_dominant, b_memory_dominant):
    B, S, d = context_state.shape
    ctx = context_state.reshape(S, d).astype(jnp.bfloat16)
    mem = memory_state.reshape(S, d).astype(jnp.bfloat16)
    ew = expert_weights.reshape(S, 4)
    biases = [b.reshape(1, d) for b in (b_conservative, b_base, b_bridge,
                                        b_memory_dominant)]
    weights = [W_conservative, W_base, W_bridge, W_memory_dominant]

    n_tiles = d // _TN
    full = lambda n: (0, 0)
    wspec = pl.BlockSpec((2 * d, _TN), lambda n: (0, n))
    bspec = pl.BlockSpec((1, _TN), lambda n: (0, n))

    in_specs = [pl.BlockSpec((S, d), full),
                pl.BlockSpec((S, d), full),
                pl.BlockSpec((S, 4), full)]
    operands = [ctx, mem, ew]
    for W, b in zip(weights, biases):
        in_specs += [wspec, bspec]
        operands += [W, b]

    out, mean_pad = pl.pallas_call(
        _fused_kernel,
        grid=(n_tiles,),
        in_specs=in_specs,
        out_specs=[pl.BlockSpec((S, _TN), lambda n: (0, n)),
                   pl.BlockSpec((8, 128), full)],
        out_shape=[jax.ShapeDtypeStruct((S, d), jnp.float32),
                   jax.ShapeDtypeStruct((8, 128), jnp.float32)],
        scratch_shapes=[pltpu.VMEM((S, 8), jnp.float32)],
        compiler_params=pltpu.CompilerParams(
            dimension_semantics=("arbitrary",)),
    )(*operands)

    fused = out.reshape(B, S, d)
    mean_weights = mean_pad[0, 0:4]
    return fused, mean_weights


# speedup vs baseline: 1.6385x; 1.6385x over previous
import jax
import jax.numpy as jnp
from jax.experimental import pallas as pl
from jax.experimental.pallas import tpu as pltpu

_S = 2048
_D = 2048
_TN = 256
_TM = 512


def _fused_kernel(ctx_ref, mem_ref, ew_ref,
                  wc_ref, bc_ref, wb_ref, bb_ref,
                  wr_ref, br_ref, wd_ref, bd_ref,
                  out_ref, mean_ref, coef_ref, acc_ref):
    n = pl.program_id(0)
    k2 = pl.program_id(1)

    @pl.when((n == 0) & (k2 == 0))
    def _routing():
        v = ew_ref[:]
        cols = [v[:, e:e + 1] for e in range(4)]
        ws = []
        for e in range(4):
            rank = jnp.zeros_like(cols[e], dtype=jnp.int32)
            for f in range(4):
                if f == e:
                    continue
                if f < e:
                    beats = cols[f] >= cols[e]
                else:
                    beats = cols[f] > cols[e]
                rank = rank + beats.astype(jnp.int32)
            sel = (rank < 2).astype(jnp.float32)
            ws.append(cols[e] * sel)
        s = ws[0] + ws[1] + ws[2] + ws[3]
        inv = 1.0 / jnp.maximum(s, 1e-8)
        w = [wi * inv for wi in ws]
        alphas = (0.4, 1.0, 0.85, 1.15)
        s_w = w[0] + w[1] + w[2] + w[3]
        coef_ref[:, 0:1] = alphas[0] * w[0]
        coef_ref[:, 1:2] = alphas[1] * w[1]
        coef_ref[:, 2:3] = alphas[2] * w[2]
        coef_ref[:, 3:4] = alphas[3] * w[3]
        coef_ref[:, 4:5] = s_w
        coef_ref[:, 5:6] = 0.15 * w[2]
        coef_ref[:, 6:8] = jnp.zeros((_S, 2), jnp.float32)
        wcat = jnp.concatenate(w, axis=1)
        m = jnp.sum(wcat, axis=0, keepdims=True) / _S
        mrow = jnp.concatenate([m, jnp.zeros((1, 124), jnp.float32)], axis=1)
        mean_ref[:] = jnp.concatenate(
            [mrow, jnp.zeros((7, 128), jnp.float32)], axis=0)

    w_refs = (wc_ref, wb_ref, wr_ref, wd_ref)
    b_refs = (bc_ref, bb_ref, br_ref, bd_ref)
    wblk = [w_refs[e][:].astype(jnp.bfloat16) for e in range(4)]

    col = pl.ds(n * _TN, _TN)
    nchunk = _S // _TM

    @pl.when(k2 == 0)
    def _k_ctx():
        for i in range(nchunk):
            rows = pl.ds(i * _TM, _TM)
            xb = ctx_ref[rows, :]
            for e in range(4):
                acc_ref[pl.ds(i * _TM + e * _S, _TM), :] = jnp.dot(
                    xb, wblk[e], preferred_element_type=jnp.float32)

    @pl.when(k2 == 1)
    def _k_mem_epilogue():
        for i in range(nchunk):
            rows = pl.ds(i * _TM, _TM)
            xb = mem_ref[rows, :]
            acc = jnp.zeros((_TM, _TN), jnp.float32)
            for e in range(4):
                h = acc_ref[pl.ds(i * _TM + e * _S, _TM), :]
                h = h + jnp.dot(xb, wblk[e],
                                preferred_element_type=jnp.float32)
                gate = jax.nn.sigmoid(h + b_refs[e][:])
                acc = acc + coef_ref[rows, e:e + 1] * gate
            ctx32 = ctx_ref[rows, col].astype(jnp.float32)
            mem32 = mem_ref[rows, col].astype(jnp.float32)
            out_ref[rows, :] = (coef_ref[rows, 4:5] * ctx32
                                + (mem32 - ctx32) * acc
                                + coef_ref[rows, 5:6] * mem32)


@jax.jit
def kernel(context_state, memory_state, expert_weights,
           W_conservative, b_conservative, W_base, b_base,
           W_bridge, b_bridge, W_memory_dominant, b_memory_dominant):
    B, S, d = context_state.shape
    ctx = context_state.reshape(S, d).astype(jnp.bfloat16)
    mem = memory_state.reshape(S, d).astype(jnp.bfloat16)
    ew = expert_weights.reshape(S, 4)
    biases = [b.reshape(1, d) for b in (b_conservative, b_base, b_bridge,
                                        b_memory_dominant)]
    weights = [W_conservative, W_base, W_bridge, W_memory_dominant]

    n_tiles = d // _TN
    full = lambda n, k: (0, 0)
    wspec = pl.BlockSpec((d, _TN), lambda n, k: (k, n))
    bspec = pl.BlockSpec((1, _TN), lambda n, k: (0, n))

    in_specs = [pl.BlockSpec((S, d), full),
                pl.BlockSpec((S, d), full),
                pl.BlockSpec((S, 4), full)]
    operands = [ctx, mem, ew]
    for W, b in zip(weights, biases):
        in_specs += [wspec, bspec]
        operands += [W, b]

    out, mean_pad = pl.pallas_call(
        _fused_kernel,
        grid=(n_tiles, 2),
        in_specs=in_specs,
        out_specs=[pl.BlockSpec((S, _TN), lambda n, k: (0, n)),
                   pl.BlockSpec((8, 128), full)],
        out_shape=[jax.ShapeDtypeStruct((S, d), jnp.float32),
                   jax.ShapeDtypeStruct((8, 128), jnp.float32)],
        scratch_shapes=[pltpu.VMEM((S, 8), jnp.float32),
                        pltpu.VMEM((4 * _S, _TN), jnp.float32)],
        compiler_params=pltpu.CompilerParams(
            dimension_semantics=("arbitrary", "arbitrary")),
    )(*operands)

    fused = out.reshape(B, S, d)
    mean_weights = mean_pad[0, 0:4]
    return fused, mean_weights
